# Initial kernel scaffold; baseline (speedup 1.0000x reference)
import jax
import jax.numpy as jnp
from jax.experimental import pallas as pl


def kernel(x, w):
    raise NotImplementedError("write your pallas kernel here")


# speedup vs baseline: 1.5679x; 1.5679x over previous
import jax
import jax.numpy as jnp
from jax import lax
from jax.experimental import pallas as pl

EMBEDDING_DIM = 64
NUM_EMBEDDINGS = 1024
ROWS_PER_TILE = 512


def _vq_body(x_ref, w_ref, o_ref):
    xb = x_ref[...]
    w = w_ref[...]
    xw = jnp.dot(xb, w)
    xsq = jnp.sum(xb * xb, axis=1, keepdims=True)
    wsq = jnp.sum(w * w, axis=0, keepdims=True)
    distances = xsq - 2.0 * xw + wsq
    idx = jnp.argmax(-distances, axis=1)
    cols = lax.broadcasted_iota(jnp.int32, distances.shape, 1)
    onehot = (idx[:, None] == cols).astype(jnp.float32)
    o_ref[...] = lax.dot_general(
        onehot, w, (((1,), (1,)), ((), ())),
        preferred_element_type=jnp.float32)


def kernel(x, w):
    flat = jnp.reshape(x, (-1, EMBEDDING_DIM))
    n = flat.shape[0]
    grid = n // ROWS_PER_TILE
    out = pl.pallas_call(
        _vq_body,
        grid=(grid,),
        in_specs=[
            pl.BlockSpec((ROWS_PER_TILE, EMBEDDING_DIM), lambda i: (i, 0)),
            pl.BlockSpec((EMBEDDING_DIM, NUM_EMBEDDINGS), lambda i: (0, 0)),
        ],
        out_specs=pl.BlockSpec((ROWS_PER_TILE, EMBEDDING_DIM), lambda i: (i, 0)),
        out_shape=jax.ShapeDtypeStruct((n, EMBEDDING_DIM), jnp.float32),
    )(flat, w)
    return jnp.reshape(out, x.shape)
